# baseline (speedup 1.0000x reference)
import functools

import jax
import jax.numpy as jnp
from jax import lax
from jax.experimental import pallas as pl
from jax.experimental.pallas import tpu as pltpu
from jax.experimental.pallas import tpu_sc as plsc

_NC = 2
_NS = 16
_L = 16


@functools.lru_cache(maxsize=None)
def _build(B, S, D):
    assert B % _L == 0
    n_chunks = B // _L
    q_per_chunk = 1
    rows_per_w = _L // q_per_chunk
    n_workers = n_chunks * q_per_chunk
    mesh = plsc.VectorSubcoreMesh(
        core_axis_name="c", subcore_axis_name="s", num_cores=1,
        num_subcores=8)

    @functools.partial(
        pl.kernel,
        mesh=mesh,
        out_type=jax.ShapeDtypeStruct((B, D), jnp.float32),
        scratch_types=[
            pltpu.VMEM((_L,), jnp.int32),
            pltpu.VMEM((_L,), jnp.int32),
            pltpu.VMEM((rows_per_w, D), jnp.float32),
            pltpu.SemaphoreType.DMA,
        ],
    )
    def gather_kernel(flat_hbm, idx_hbm, out_hbm, ids_v, gids_v, rows_v, sem):
        wid = lax.axis_index("s")

        @pl.when(wid < n_workers)
        def _():
            chunk = wid // q_per_chunk
            q = wid % q_per_chunk
            base = chunk * _L
            with jax.named_scope("idx_load"):
                pltpu.sync_copy(idx_hbm.at[pl.ds(base, _L)], ids_v)
                lane = lax.iota(jnp.int32, _L)
                gids_v[...] = (lane + base) * S + ids_v[...]
            with jax.named_scope("row_gather"):
                pltpu.async_copy(
                    flat_hbm.at[gids_v.at[pl.ds(q * rows_per_w, rows_per_w)]],
                    rows_v,
                    sem,
                ).wait()
            with jax.named_scope("writeback"):
                pltpu.sync_copy(
                    rows_v,
                    out_hbm.at[pl.ds(base + q * rows_per_w, rows_per_w)])

    return gather_kernel


def kernel(hidden_states, input_id):
    B, S, D = hidden_states.shape
    flat = hidden_states.reshape(B * S, D)
    return _build(B, S, D)(flat, input_id.astype(jnp.int32))


# speedup vs baseline: 1.0227x; 1.0227x over previous
import functools

import jax
import jax.numpy as jnp
from jax import lax
from jax.experimental import pallas as pl
from jax.experimental.pallas import tpu as pltpu
from jax.experimental.pallas import tpu_sc as plsc

_NC = 2
_NS = 16
_L = 16


@functools.lru_cache(maxsize=None)
def _build(B, S, D):
    assert B % _L == 0
    n_chunks = B // _L
    q_per_chunk = 2
    rows_per_w = _L // q_per_chunk
    n_workers = n_chunks * q_per_chunk
    mesh = plsc.VectorSubcoreMesh(
        core_axis_name="c", subcore_axis_name="s", num_cores=1)

    @functools.partial(
        pl.kernel,
        mesh=mesh,
        out_type=jax.ShapeDtypeStruct((B, D), jnp.float32),
        scratch_types=[
            pltpu.VMEM((_L,), jnp.int32),
            pltpu.VMEM((_L,), jnp.int32),
            pltpu.VMEM((rows_per_w, D), jnp.float32),
            pltpu.SemaphoreType.DMA,
        ],
    )
    def gather_kernel(flat_hbm, idx_hbm, out_hbm, ids_v, gids_v, rows_v, sem):
        wid = lax.axis_index("s")
        chunk = wid // q_per_chunk
        q = wid % q_per_chunk
        base = chunk * _L
        pltpu.sync_copy(idx_hbm.at[pl.ds(base, _L)], ids_v)
        lane = lax.iota(jnp.int32, _L)
        gids_v[...] = (lane + base) * S + ids_v[...]
        pltpu.async_copy(
            flat_hbm.at[gids_v.at[pl.ds(q * rows_per_w, rows_per_w)]],
            rows_v,
            sem,
        ).wait()
        pltpu.sync_copy(
            rows_v, out_hbm.at[pl.ds(base + q * rows_per_w, rows_per_w)])

    return gather_kernel


def kernel(hidden_states, input_id):
    B, S, D = hidden_states.shape
    flat = hidden_states.reshape(B * S, D)
    return _build(B, S, D)(flat, input_id.astype(jnp.int32))


# speedup vs baseline: 1.0237x; 1.0009x over previous
import functools

import jax
import jax.numpy as jnp
from jax import lax
from jax.experimental import pallas as pl
from jax.experimental.pallas import tpu as pltpu
from jax.experimental.pallas import tpu_sc as plsc

_NC = 2
_NS = 16
_L = 16


@functools.lru_cache(maxsize=None)
def _build(B, S, D):
    assert B % _L == 0
    n_chunks = B // _L
    q_per_chunk = 2
    rows_per_w = _L // q_per_chunk
    n_workers = n_chunks * q_per_chunk
    mesh = plsc.VectorSubcoreMesh(
        core_axis_name="c", subcore_axis_name="s", num_cores=1)

    @functools.partial(
        pl.kernel,
        mesh=mesh,
        out_type=jax.ShapeDtypeStruct((B, D), jnp.float32),
        compiler_params=pltpu.CompilerParams(skip_device_barrier=True),
        scratch_types=[
            pltpu.VMEM((_L,), jnp.int32),
            pltpu.VMEM((_L,), jnp.int32),
            pltpu.VMEM((rows_per_w, D), jnp.float32),
            pltpu.SemaphoreType.DMA,
        ],
    )
    def gather_kernel(flat_hbm, idx_hbm, out_hbm, ids_v, gids_v, rows_v, sem):
        wid = lax.axis_index("s")
        chunk = wid // q_per_chunk
        q = wid % q_per_chunk
        base = chunk * _L
        pltpu.sync_copy(idx_hbm.at[pl.ds(base, _L)], ids_v)
        lane = lax.iota(jnp.int32, _L)
        gids_v[...] = (lane + base) * S + ids_v[...]
        pltpu.async_copy(
            flat_hbm.at[gids_v.at[pl.ds(q * rows_per_w, rows_per_w)]],
            rows_v,
            sem,
        ).wait()
        pltpu.sync_copy(
            rows_v, out_hbm.at[pl.ds(base + q * rows_per_w, rows_per_w)])

    return gather_kernel


def kernel(hidden_states, input_id):
    B, S, D = hidden_states.shape
    flat = hidden_states.reshape(B * S, D)
    return _build(B, S, D)(flat, input_id.astype(jnp.int32))
